# baseline (speedup 1.0000x reference)
import jax
import jax.numpy as jnp
from jax.experimental import pallas as pl
from jax.experimental.pallas import tpu as pltpu


def _conv_body(H, W, Ho, Wo, Wp, s, kernel):
    KH, KW = kernel

    def body(x_ref, wt_ref, b_ref, o_ref, xp_ref, cols_ref):
        G = x_ref.shape[0]
        C = x_ref.shape[2]
        L = Ho * Wo
        for g in range(G):
            xt = x_ref[g].reshape(H, W, C)
            xp_ref[0] = jnp.zeros((Wp, C), jnp.float32)
            xp_ref[:, 0, :] = jnp.zeros((H + 2, C), jnp.float32)
            xp_ref[1:H + 1, 1:W + 1, :] = xt
            for kh in range(KH):
                for kw in range(KW):
                    t = kh * KW + kw
                    patch = xp_ref[pl.ds(kh, Ho, s), pl.ds(kw, Wo, s), :]
                    cols_ref[g * L:(g + 1) * L, t * C:(t + 1) * C] = \
                        patch.reshape(L, C).astype(cols_ref.dtype)
        acc = jnp.dot(cols_ref[...], wt_ref[...],
                      preferred_element_type=jnp.float32)
        acc = (acc + b_ref[...]).astype(o_ref.dtype)
        o_ref[...] = acc.reshape(o_ref.shape)

    return body


def kernel(x, w, b):
    N, C, H, W = x.shape
    Cout, Cin, KH, KW = w.shape
    assert Cin == C
    s, padding = 2, 1
    Ho = (H + 2 * padding - KH) // s + 1
    Wo = (W + 2 * padding - KW) // s + 1
    L = Ho * Wo
    K = C * KH * KW
    Wp = ((W + 2 + 7) // 8) * 8

    xt_all = jnp.transpose(x, (0, 2, 3, 1)).reshape(N, H * W, C)
    wt = w.transpose(2, 3, 1, 0).reshape(K, Cout).astype(jnp.bfloat16)
    brow = b.reshape(1, Cout)

    G = 4 if N % 4 == 0 else (2 if N % 2 == 0 else 1)
    out = pl.pallas_call(
        _conv_body(H, W, Ho, Wo, Wp, s, (KH, KW)),
        out_shape=jax.ShapeDtypeStruct((N, L, Cout), x.dtype),
        grid=(N // G,),
        in_specs=[
            pl.BlockSpec((G, H * W, C), lambda n: (n, 0, 0)),
            pl.BlockSpec((K, Cout), lambda n: (0, 0)),
            pl.BlockSpec((1, Cout), lambda n: (0, 0)),
        ],
        out_specs=pl.BlockSpec((G, L, Cout), lambda n: (n, 0, 0)),
        scratch_shapes=[
            pltpu.VMEM((H + 2, Wp, C), jnp.float32),
            pltpu.VMEM((G * L, K), jnp.bfloat16),
        ],
        compiler_params=pltpu.CompilerParams(
            dimension_semantics=("parallel",)),
    )(xt_all, wt, brow)

    return out.transpose(0, 2, 1).reshape(N, Cout, Ho, Wo)


# speedup vs baseline: 1.1267x; 1.1267x over previous
import jax
import jax.numpy as jnp
from jax.experimental import pallas as pl
from jax.experimental.pallas import tpu as pltpu


def _conv_body(H, W, Ho, Wo, Wp, s, kernel):
    KH, KW = kernel

    def body(x_ref, wt_ref, b_ref, o_ref, xp_ref, cols_ref):
        G = x_ref.shape[0]
        C = x_ref.shape[2]
        L = Ho * Wo
        xp_ref[0] = jnp.zeros((Wp, C), jnp.float32)
        xp_ref[:, 0, :] = jnp.zeros((H + 2, C), jnp.float32)
        for g in range(G):
            xp_ref[1:H + 1, 1:W + 1, :] = x_ref[g].reshape(H, W, C)
            for kh in range(KH):
                for kw in range(KW):
                    t = kh * KW + kw
                    patch = xp_ref[pl.ds(kh, Ho, s), pl.ds(kw, Wo, s), :]
                    cols_ref[g * L:(g + 1) * L, t * C:(t + 1) * C] = \
                        patch.reshape(L, C)
        acc = jnp.dot(cols_ref[...], wt_ref[...],
                      preferred_element_type=jnp.float32)
        acc = (acc + b_ref[...]).astype(o_ref.dtype)
        o_ref[...] = acc.reshape(o_ref.shape)

    return body


def kernel(x, w, b):
    N, C, H, W = x.shape
    Cout, Cin, KH, KW = w.shape
    assert Cin == C
    s, padding = 2, 1
    Ho = (H + 2 * padding - KH) // s + 1
    Wo = (W + 2 * padding - KW) // s + 1
    L = Ho * Wo
    K = C * KH * KW
    Wp = ((W + 2 + 7) // 8) * 8

    xt_all = jnp.transpose(x, (0, 2, 3, 1)).reshape(N, H * W, C)
    wt = w.transpose(2, 3, 1, 0).reshape(K, Cout)
    brow = b.reshape(1, Cout)

    G = 4 if N % 4 == 0 else (2 if N % 2 == 0 else 1)
    out = pl.pallas_call(
        _conv_body(H, W, Ho, Wo, Wp, s, (KH, KW)),
        out_shape=jax.ShapeDtypeStruct((N, L, Cout), x.dtype),
        grid=(N // G,),
        in_specs=[
            pl.BlockSpec((G, H * W, C), lambda n: (n, 0, 0)),
            pl.BlockSpec((K, Cout), lambda n: (0, 0)),
            pl.BlockSpec((1, Cout), lambda n: (0, 0)),
        ],
        out_specs=pl.BlockSpec((G, L, Cout), lambda n: (n, 0, 0)),
        scratch_shapes=[
            pltpu.VMEM((H + 2, Wp, C), jnp.float32),
            pltpu.VMEM((G * L, K), jnp.float32),
        ],
        compiler_params=pltpu.CompilerParams(
            dimension_semantics=("parallel",)),
    )(xt_all, wt, brow)

    return out.transpose(0, 2, 1).reshape(N, Cout, Ho, Wo)
